# baseline (speedup 1.0000x reference)
import functools

import jax
import jax.numpy as jnp
from jax import lax
from jax.experimental import pallas as pl
from jax.experimental.pallas import tpu as pltpu
from jax.experimental.pallas import tpu_sc as plsc

NC = 2
NS = 16
NW = NC * NS
D = 32
G = 128

B = 16384 * 20
B_PER_W = B // NW
GROUPS_PER_W = B_PER_W // G
CHUNK_GROUPS = 4
CHUNK_ROWS = CHUNK_GROUPS * G
NCHUNKS = GROUPS_PER_W // CHUNK_GROUPS

_mesh = plsc.VectorSubcoreMesh(core_axis_name="c", subcore_axis_name="s")


@functools.partial(
    pl.kernel,
    mesh=_mesh,
    out_type=jax.ShapeDtypeStruct((D, B), jnp.float32),
    scratch_types=[
        pltpu.VMEM((GROUPS_PER_W, G), jnp.int32),
        pltpu.VMEM((CHUNK_ROWS, D), jnp.float32),
        pltpu.VMEM((CHUNK_ROWS, D), jnp.float32),
        pltpu.VMEM((D, CHUNK_ROWS), jnp.float32),
        pltpu.VMEM((D, CHUNK_ROWS), jnp.float32),
        pltpu.SemaphoreType.DMA,
        pltpu.SemaphoreType.DMA,
        pltpu.SemaphoreType.DMA,
        pltpu.SemaphoreType.DMA,
    ],
    compiler_params=pltpu.CompilerParams(
        use_tc_tiling_on_sc=False, needs_layout_passes=False
    ),
)
def _gather_kernel(t_hbm, idx_hbm, out_hbm, idx_v, buf0, buf1, tbuf0, tbuf1,
                   gsem0, gsem1, osem0, osem1):
    wid = lax.axis_index("s") * NC + lax.axis_index("c")
    row0 = wid * B_PER_W
    bufs = (buf0, buf1)
    tbufs = (tbuf0, tbuf1)
    gsems = (gsem0, gsem1)
    osems = (osem0, osem1)

    pltpu.sync_copy(idx_hbm.at[pl.ds(wid * GROUPS_PER_W, GROUPS_PER_W)], idx_v)

    def fire_gathers(c, b):
        for j in range(CHUNK_GROUPS):
            pltpu.async_copy(
                t_hbm.at[idx_v.at[c * CHUNK_GROUPS + j]],
                bufs[b].at[pl.ds(j * G, G)],
                gsems[b],
            )

    def drain_gathers(b):
        for j in range(CHUNK_GROUPS):
            pltpu.make_async_copy(
                t_hbm.at[pl.ds(0, G)],
                bufs[b].at[pl.ds(j * G, G)],
                gsems[b],
            ).wait()

    def transpose_chunk(b):
        def rb_body(rb, carry):
            vrow = rb * 16 + lax.iota(jnp.int32, 16)
            for cc in range(D):
                vcc = jnp.full((16,), cc, jnp.int32)
                v = plsc.load_gather(bufs[b], [vrow, vcc])
                tbufs[b][cc, pl.ds(rb * 16, 16)] = v
            return carry

        lax.fori_loop(0, CHUNK_ROWS // 16, rb_body, 0)

    def fire_out(c, b):
        pltpu.async_copy(
            tbufs[b],
            out_hbm.at[:, pl.ds(row0 + c * CHUNK_ROWS, CHUNK_ROWS)],
            osems[b],
        )

    def drain_out(b):
        pltpu.make_async_copy(
            out_hbm.at[:, pl.ds(0, CHUNK_ROWS)],
            tbufs[b],
            osems[b],
        ).wait()

    fire_gathers(0, 0)
    fire_gathers(1, 1)
    for c in (0, 1):
        b = c
        drain_gathers(b)
        transpose_chunk(b)
        fire_out(c, b)
        fire_gathers(c + 2, b)

    def pair_body(p, carry):
        for b in range(2):
            c = 2 * p + b
            drain_gathers(b)
            drain_out(b)
            transpose_chunk(b)
            fire_out(c, b)
            fire_gathers(c + 2, b)
        return carry

    lax.fori_loop(1, (NCHUNKS - 2) // 2, pair_body, 0)

    for c in (NCHUNKS - 2, NCHUNKS - 1):
        b = c % 2
        drain_gathers(b)
        drain_out(b)
        transpose_chunk(b)
        fire_out(c, b)
    drain_out(0)
    drain_out(1)


def kernel(t, idx):
    idx32 = idx.astype(jnp.int32).reshape(B // G, G)
    out_cm = _gather_kernel(t, idx32)
    return out_cm.T.reshape(idx.shape[0], idx.shape[1], D)


# speedup vs baseline: 2.2892x; 2.2892x over previous
import functools

import jax
import jax.numpy as jnp
from jax import lax
from jax.experimental import pallas as pl
from jax.experimental.pallas import tpu as pltpu
from jax.experimental.pallas import tpu_sc as plsc

NC = 2
NS = 16
NW = NC * NS
D = 32
G = 128
K = 20
NB = 16384 // G
BLOCKS_PER_W = NB // NW
UNITS_PER_W = K * BLOCKS_PER_W
CHUNK_UNITS = 4
NCHUNKS = UNITS_PER_W // CHUNK_UNITS
TP = 133

_mesh = plsc.VectorSubcoreMesh(core_axis_name="c", subcore_axis_name="s")


@functools.partial(
    pl.kernel,
    mesh=_mesh,
    out_type=jax.ShapeDtypeStruct((K, D // 8, G, 8, G), jnp.float32),
    scratch_types=[
        pltpu.VMEM((UNITS_PER_W, G), jnp.int32),
        pltpu.VMEM((CHUNK_UNITS * G, D), jnp.float32),
        pltpu.VMEM((CHUNK_UNITS * G, D), jnp.float32),
        pltpu.VMEM((CHUNK_UNITS, D // 8, 8, TP), jnp.float32),
        pltpu.VMEM((CHUNK_UNITS, D // 8, 8, TP), jnp.float32),
        pltpu.SemaphoreType.DMA,
        pltpu.SemaphoreType.DMA,
        pltpu.SemaphoreType.DMA,
        pltpu.SemaphoreType.DMA,
    ],
    compiler_params=pltpu.CompilerParams(
        use_tc_tiling_on_sc=False, needs_layout_passes=False
    ),
)
def _gather_kernel(t_hbm, idx_hbm, out_hbm, idx_v, buf0, buf1, tbuf0, tbuf1,
                   gsem0, gsem1, osem0, osem1):
    wid = lax.axis_index("s") * NC + lax.axis_index("c")
    bufs = (buf0, buf1)
    tbufs = (tbuf0, tbuf1)
    gsems = (gsem0, gsem1)
    osems = (osem0, osem1)

    pltpu.sync_copy(idx_hbm.at[pl.ds(wid * UNITS_PER_W, UNITS_PER_W)], idx_v)

    iota = lax.iota(jnp.int32, 16)
    vi_lo = iota // 8
    vs = iota % 8
    two = jnp.full((16,), 2, jnp.int32)

    def fire_gathers(c, b):
        for q in range(CHUNK_UNITS):
            pltpu.async_copy(
                t_hbm.at[idx_v.at[c * CHUNK_UNITS + q]],
                bufs[b].at[pl.ds(q * G, G)],
                gsems[b],
            )

    def drain_gathers(b):
        for q in range(CHUNK_UNITS):
            pltpu.make_async_copy(
                t_hbm.at[pl.ds(0, G)],
                bufs[b].at[pl.ds(q * G, G)],
                gsems[b],
            ).wait()

    def transpose_chunk(b):
        def row_body(r, carry):
            vr = jnp.full((16,), 0, jnp.int32) + r
            for q in range(CHUNK_UNITS):
                vq = jnp.full((16,), q, jnp.int32)
                lo = bufs[b][q * G + r, pl.ds(0, 16)]
                hi = bufs[b][q * G + r, pl.ds(16, 16)]
                plsc.store_scatter(tbufs[b], [vq, vi_lo, vs, vr], lo)
                plsc.store_scatter(tbufs[b], [vq, vi_lo + two, vs, vr], hi)
            return carry

        lax.fori_loop(0, G, row_body, 0)

    def fire_out(c, b):
        for q in range(CHUNK_UNITS):
            pltpu.async_copy(
                tbufs[b].at[q, :, :, pl.ds(0, G)],
                out_hbm.at[c, :, wid * BLOCKS_PER_W + q, :, :],
                osems[b],
            )

    def drain_out(b):
        for q in range(CHUNK_UNITS):
            pltpu.make_async_copy(
                out_hbm.at[0, :, 0, :, :],
                tbufs[b].at[q, :, :, pl.ds(0, G)],
                osems[b],
            ).wait()

    fire_gathers(0, 0)
    fire_gathers(1, 1)
    for c in (0, 1):
        b = c
        drain_gathers(b)
        transpose_chunk(b)
        fire_out(c, b)
        fire_gathers(c + 2, b)

    def pair_body(p, carry):
        for b in range(2):
            c = 2 * p + b
            drain_gathers(b)
            drain_out(b)
            transpose_chunk(b)
            fire_out(c, b)
            fire_gathers(c + 2, b)
        return carry

    lax.fori_loop(1, (NCHUNKS - 2) // 2, pair_body, 0)

    for c in (NCHUNKS - 2, NCHUNKS - 1):
        b = c % 2
        drain_gathers(b)
        drain_out(b)
        transpose_chunk(b)
        fire_out(c, b)
    drain_out(0)
    drain_out(1)


def kernel(t, idx):
    idx32 = idx.astype(jnp.int32)
    a = idx32.reshape(NW, BLOCKS_PER_W, G, K)
    a = a.transpose(0, 3, 1, 2)
    idx_arr = a.reshape(NW * UNITS_PER_W, G)
    out5 = _gather_kernel(t, idx_arr)
    return out5.transpose(2, 4, 0, 1, 3).reshape(16384, K, D)
